# baseline (speedup 1.0000x reference)
import functools

import jax
import jax.numpy as jnp
from jax import lax
from jax.experimental import pallas as pl
from jax.experimental.pallas import tpu as pltpu
from jax.experimental.pallas import tpu_sc as plsc

E_PAD = 128
TILE = 256


def _router_body(x_ref, gwt_ref, sgw_ref, pos0_ref, pos1_ref, wa_ref, wb_ref,
                 te_ref, gsh_ref):
    logits = jnp.dot(x_ref[...], gwt_ref[...], preferred_element_type=jnp.float32)
    T = logits.shape[0]
    gsh_ref[...] = jax.nn.sigmoid(
        jnp.sum(x_ref[...] * sgw_ref[...], axis=1, keepdims=True))
    lane = jax.lax.broadcasted_iota(jnp.int32, (T, E_PAD), 1)
    neg = jnp.float32(-1e30)
    logits = jnp.where(lane < 8, logits, neg)
    m1 = jnp.max(logits, axis=1, keepdims=True)
    i1 = jnp.min(jnp.where(logits == m1, lane, E_PAD), axis=1, keepdims=True)
    masked = jnp.where(lane == i1, neg, logits)
    m2 = jnp.max(masked, axis=1, keepdims=True)
    i2 = jnp.min(jnp.where(masked == m2, lane, E_PAD), axis=1, keepdims=True)
    w1 = 1.0 / (1.0 + jnp.exp(m2 - m1))
    wa_ref[...] = jnp.broadcast_to(w1, wa_ref.shape)
    wb_ref[...] = jnp.broadcast_to(1.0 - w1, wb_ref.shape)

    cnt = (jnp.where(lane == i1, 1.0, 0.0) + jnp.where(lane == i2, 1.0, 0.0))
    ri = jax.lax.broadcasted_iota(jnp.int32, (T, T), 0)
    ci = jax.lax.broadcasted_iota(jnp.int32, (T, T), 1)
    ltri = jnp.where(ci <= ri, 1.0, 0.0).astype(jnp.bfloat16)
    rank_incl = jnp.dot(ltri, cnt.astype(jnp.bfloat16),
                        preferred_element_type=jnp.float32)
    rank0 = rank_incl - cnt
    totals = rank_incl[T - 1:T, :]
    padded = jnp.ceil(totals * (1.0 / TILE)) * TILE
    r128 = jax.lax.broadcasted_iota(jnp.int32, (E_PAD, E_PAD), 0)
    c128 = jax.lax.broadcasted_iota(jnp.int32, (E_PAD, E_PAD), 1)
    upper = jnp.where(r128 < c128, 1.0, 0.0).astype(jnp.bfloat16)
    starts = jnp.dot(padded.astype(jnp.bfloat16), upper,
                     preferred_element_type=jnp.float32)
    ends = starts + padded
    pos0 = jnp.sum(jnp.where(lane == i1, starts + rank0, 0.0), axis=1,
                   keepdims=True)
    pos1 = jnp.sum(jnp.where(lane == i2, starts + rank0, 0.0), axis=1,
                   keepdims=True)
    pos0_ref[...] = pos0.astype(jnp.int32)
    pos1_ref[...] = pos1.astype(jnp.int32)
    rt = jax.lax.broadcasted_iota(jnp.int32, (E_PAD, E_PAD), 0).astype(jnp.float32) * TILE
    elig = (c128 < 8) & (jnp.broadcast_to(ends, (E_PAD, E_PAD)) <= rt)
    te = jnp.sum(jnp.where(elig, 1, 0), axis=1, keepdims=True)
    te_ref[...] = jnp.minimum(te, 7).astype(jnp.int32)


def _grouped_body(te_ref, xs_ref, wg_ref, wu_ref, wd_ref, ys_ref):
    x = xs_ref[...]
    g = jnp.dot(x, wg_ref[0], preferred_element_type=jnp.float32)
    u = jnp.dot(x, wu_ref[0], preferred_element_type=jnp.float32)
    a = (jax.nn.silu(g) * u).astype(jnp.bfloat16)
    ys_ref[...] = jnp.dot(a, wd_ref[0], preferred_element_type=jnp.float32)


def _shared_act_body(x_ref, wsg_ref, wsu_ref, a_ref):
    x = x_ref[...]
    g = jnp.dot(x, wsg_ref[...], preferred_element_type=jnp.float32)
    u = jnp.dot(x, wsu_ref[...], preferred_element_type=jnp.float32)
    a_ref[...] = (jax.nn.silu(g) * u).astype(jnp.bfloat16)


def _shared_down_body(a_ref, wsd_ref, gsh_ref, out_ref):
    out_ref[...] = gsh_ref[...] * jnp.dot(a_ref[...], wsd_ref[...],
                                          preferred_element_type=jnp.float32)


def _make_dispatch(T, D, NP, NW, CH):
    mesh = plsc.VectorSubcoreMesh(core_axis_name="c", subcore_axis_name="s")
    tpw = T // NW

    @functools.partial(
        pl.kernel, mesh=mesh,
        out_type=jax.ShapeDtypeStruct((NP, D), jnp.int32),
        scratch_types=[
            pltpu.VMEM((CH, D), jnp.int32),
            pltpu.VMEM((CH,), jnp.int32),
            pltpu.VMEM((CH,), jnp.int32),
            pltpu.SemaphoreType.DMA,
        ],
    )
    def dispatch(x_hbm, p0_hbm, p1_hbm, xs_hbm, rows_v, i0_v, i1_v, sem):
        wid = lax.axis_index("s") * 2 + lax.axis_index("c")
        base = wid * tpw
        for ch in range(0, tpw, CH):
            b = base + ch
            pltpu.sync_copy(x_hbm.at[pl.ds(b, CH)], rows_v)
            pltpu.sync_copy(p0_hbm.at[pl.ds(b, CH)], i0_v)
            pltpu.sync_copy(p1_hbm.at[pl.ds(b, CH)], i1_v)
            c0 = pltpu.async_copy(rows_v, xs_hbm.at[i0_v], sem)
            c1 = pltpu.async_copy(rows_v, xs_hbm.at[i1_v], sem)
            c0.wait()
            c1.wait()

    return dispatch


def _make_combine(T, D, NW, CH):
    mesh = plsc.VectorSubcoreMesh(core_axis_name="c", subcore_axis_name="s")
    tpw = T // NW
    d16 = D // 16

    @functools.partial(
        pl.kernel, mesh=mesh,
        out_type=jax.ShapeDtypeStruct((T, D), jnp.float32),
        scratch_types=[
            pltpu.VMEM((CH, D), jnp.float32),
            pltpu.VMEM((CH, D), jnp.float32),
            pltpu.VMEM((CH, D), jnp.float32),
            pltpu.VMEM((tpw,), jnp.int32),
            pltpu.VMEM((tpw,), jnp.int32),
            pltpu.VMEM((tpw, 16), jnp.float32),
            pltpu.VMEM((tpw, 16), jnp.float32),
            pltpu.SemaphoreType.DMA,
        ],
    )
    def combine(ys_hbm, p0_hbm, p1_hbm, wa_hbm, wb_hbm, sh_hbm, out_hbm,
                y0_v, y1_v, sh_v, p0_v, p1_v, wa_v, wb_v, sem):
        wid = lax.axis_index("s") * 2 + lax.axis_index("c")
        base = wid * tpw
        pltpu.sync_copy(p0_hbm.at[pl.ds(base, tpw)], p0_v)
        pltpu.sync_copy(p1_hbm.at[pl.ds(base, tpw)], p1_v)
        pltpu.sync_copy(wa_hbm.at[pl.ds(base, tpw)], wa_v)
        pltpu.sync_copy(wb_hbm.at[pl.ds(base, tpw)], wb_v)
        for ch in range(0, tpw, CH):
            b = base + ch
            pltpu.async_copy(ys_hbm.at[p0_v.at[pl.ds(ch, CH)]], y0_v, sem).wait()
            pltpu.async_copy(ys_hbm.at[p1_v.at[pl.ds(ch, CH)]], y1_v, sem).wait()
            pltpu.sync_copy(sh_hbm.at[pl.ds(b, CH)], sh_v)

            for r in range(CH):
                was = wa_v[ch + r, :]
                wbs = wb_v[ch + r, :]

                def body(c, _, r=r, was=was, wbs=wbs):
                    o = c * 16
                    v = (was * y0_v[r, pl.ds(o, 16)]
                         + wbs * y1_v[r, pl.ds(o, 16)]
                         + sh_v[r, pl.ds(o, 16)])
                    y0_v[r, pl.ds(o, 16)] = v
                    return 0

                lax.fori_loop(0, d16, body, 0)
            pltpu.sync_copy(y0_v, out_hbm.at[pl.ds(b, CH)])

    return combine


def _pad_ff(w, axis, ffp):
    pad = [(0, 0)] * w.ndim
    pad[axis] = (0, ffp - w.shape[axis])
    return jnp.pad(w, pad)


def kernel(hidden_states, gate_w, w_gate, w_up, w_down, ws_gate, ws_up,
           ws_down, shared_gate_w):
    orig_shape = hidden_states.shape
    D = orig_shape[-1]
    x = hidden_states.reshape(-1, D)
    T = x.shape[0]
    E, _, FF = w_gate.shape
    FFS = ws_gate.shape[1]
    NT = (2 * T) // TILE + E
    NP = NT * TILE

    gwt = jnp.zeros((D, E_PAD), jnp.float32).at[:, :E].set(gate_w.T)
    sgw = shared_gate_w.astype(jnp.float32).reshape(1, D)
    pos0, pos1, wa, wb, te, gsh = pl.pallas_call(
        _router_body,
        out_shape=(
            jax.ShapeDtypeStruct((T, 1), jnp.int32),
            jax.ShapeDtypeStruct((T, 1), jnp.int32),
            jax.ShapeDtypeStruct((T, 16), jnp.float32),
            jax.ShapeDtypeStruct((T, 16), jnp.float32),
            jax.ShapeDtypeStruct((E_PAD, 1), jnp.int32),
            jax.ShapeDtypeStruct((T, 1), jnp.float32),
        ),
    )(x, gwt, sgw)
    pos0 = pos0.reshape(T)
    pos1 = pos1.reshape(T)
    te = te.reshape(E_PAD)

    xb = x.astype(jnp.bfloat16)
    x_words = jax.lax.bitcast_convert_type(
        xb.reshape(T, D // 2, 2), jnp.int32)
    xs_words = _make_dispatch(T, D // 2, NP, 32, 16)(x_words, pos0, pos1)
    xs = jax.lax.bitcast_convert_type(xs_words, jnp.bfloat16).reshape(NP, D)

    FFP = ((FF + 511) // 512) * 512
    wg = _pad_ff(w_gate.astype(jnp.bfloat16), 2, FFP)
    wu = _pad_ff(w_up.astype(jnp.bfloat16), 2, FFP)
    wd = _pad_ff(w_down.astype(jnp.bfloat16), 1, FFP)
    ys = pl.pallas_call(
        _grouped_body,
        grid_spec=pltpu.PrefetchScalarGridSpec(
            num_scalar_prefetch=1,
            grid=(NT,),
            in_specs=[
                pl.BlockSpec((TILE, D), lambda i, te_ref: (i, 0)),
                pl.BlockSpec((1, D, FFP), lambda i, te_ref: (te_ref[i], 0, 0)),
                pl.BlockSpec((1, D, FFP), lambda i, te_ref: (te_ref[i], 0, 0)),
                pl.BlockSpec((1, FFP, D), lambda i, te_ref: (te_ref[i], 0, 0)),
            ],
            out_specs=pl.BlockSpec((TILE, D), lambda i, te_ref: (i, 0)),
        ),
        out_shape=jax.ShapeDtypeStruct((NP, D), jnp.float32),
    )(te, xs, wg, wu, wd)

    cs = 512 if FFS % 512 == 0 else FFS
    nj = FFS // cs
    wsg = ws_gate.astype(jnp.bfloat16)
    wsu = ws_up.astype(jnp.bfloat16)
    wsd = ws_down.astype(jnp.bfloat16)
    act = pl.pallas_call(
        _shared_act_body,
        grid=(nj,),
        in_specs=[
            pl.BlockSpec((T, D), lambda j: (0, 0)),
            pl.BlockSpec((D, cs), lambda j: (0, j)),
            pl.BlockSpec((D, cs), lambda j: (0, j)),
        ],
        out_specs=pl.BlockSpec((T, cs), lambda j: (0, j)),
        out_shape=jax.ShapeDtypeStruct((T, FFS), jnp.bfloat16),
    )(xb, wsg, wsu)
    cd = 512 if D % 512 == 0 else D
    nd = D // cd
    sh = pl.pallas_call(
        _shared_down_body,
        grid=(nd,),
        in_specs=[
            pl.BlockSpec((T, FFS), lambda n: (0, 0)),
            pl.BlockSpec((FFS, cd), lambda n: (0, n)),
            pl.BlockSpec((T, 1), lambda n: (0, 0)),
        ],
        out_specs=pl.BlockSpec((T, cd), lambda n: (0, n)),
        out_shape=jax.ShapeDtypeStruct((T, D), jnp.float32),
    )(act, wsd, gsh)

    final = _make_combine(T, D, 32, 8)(ys, pos0, pos1, wa, wb, sh)
    return final.reshape(orig_shape)


# speedup vs baseline: 1.4424x; 1.4424x over previous
import functools

import jax
import jax.numpy as jnp
from jax import lax
from jax.experimental import pallas as pl
from jax.experimental.pallas import tpu as pltpu
from jax.experimental.pallas import tpu_sc as plsc

E_PAD = 128
TILE = 256


def _router_body(x_ref, gwt_ref, sgw_ref, pos0_ref, pos1_ref, wa_ref, wb_ref,
                 te_ref, gsh_ref):
    logits = jnp.dot(x_ref[...], gwt_ref[...], preferred_element_type=jnp.float32)
    T = logits.shape[0]
    gsh_ref[...] = jax.nn.sigmoid(
        jnp.sum(x_ref[...] * sgw_ref[...], axis=1, keepdims=True))
    lane = jax.lax.broadcasted_iota(jnp.int32, (T, E_PAD), 1)
    neg = jnp.float32(-1e30)
    logits = jnp.where(lane < 8, logits, neg)
    m1 = jnp.max(logits, axis=1, keepdims=True)
    i1 = jnp.min(jnp.where(logits == m1, lane, E_PAD), axis=1, keepdims=True)
    masked = jnp.where(lane == i1, neg, logits)
    m2 = jnp.max(masked, axis=1, keepdims=True)
    i2 = jnp.min(jnp.where(masked == m2, lane, E_PAD), axis=1, keepdims=True)
    w1 = 1.0 / (1.0 + jnp.exp(m2 - m1))
    wa_ref[...] = jnp.broadcast_to(w1, wa_ref.shape)
    wb_ref[...] = jnp.broadcast_to(1.0 - w1, wb_ref.shape)

    cnt = (jnp.where(lane == i1, 1.0, 0.0) + jnp.where(lane == i2, 1.0, 0.0))
    ri = jax.lax.broadcasted_iota(jnp.int32, (T, T), 0)
    ci = jax.lax.broadcasted_iota(jnp.int32, (T, T), 1)
    ltri = jnp.where(ci <= ri, 1.0, 0.0).astype(jnp.bfloat16)
    rank_incl = jnp.dot(ltri, cnt.astype(jnp.bfloat16),
                        preferred_element_type=jnp.float32)
    rank0 = rank_incl - cnt
    totals = rank_incl[T - 1:T, :]
    padded = jnp.ceil(totals * (1.0 / TILE)) * TILE
    r128 = jax.lax.broadcasted_iota(jnp.int32, (E_PAD, E_PAD), 0)
    c128 = jax.lax.broadcasted_iota(jnp.int32, (E_PAD, E_PAD), 1)
    upper = jnp.where(r128 < c128, 1.0, 0.0).astype(jnp.bfloat16)
    starts = jnp.dot(padded.astype(jnp.bfloat16), upper,
                     preferred_element_type=jnp.float32)
    ends = starts + padded
    pos0 = jnp.sum(jnp.where(lane == i1, starts + rank0, 0.0), axis=1,
                   keepdims=True)
    pos1 = jnp.sum(jnp.where(lane == i2, starts + rank0, 0.0), axis=1,
                   keepdims=True)
    pos0_ref[...] = pos0.astype(jnp.int32)
    pos1_ref[...] = pos1.astype(jnp.int32)
    rt = jax.lax.broadcasted_iota(jnp.int32, (E_PAD, E_PAD), 0).astype(jnp.float32) * TILE
    elig = (c128 < 8) & (jnp.broadcast_to(ends, (E_PAD, E_PAD)) <= rt)
    te = jnp.sum(jnp.where(elig, 1, 0), axis=1, keepdims=True)
    te_ref[...] = jnp.minimum(te, 7).astype(jnp.int32)


def _grouped_body(te_ref, xs_ref, wg_ref, wu_ref, wd_ref, ys_ref):
    x = xs_ref[...].astype(jnp.bfloat16)
    g = jnp.dot(x, wg_ref[0], preferred_element_type=jnp.float32)
    u = jnp.dot(x, wu_ref[0], preferred_element_type=jnp.float32)
    a = (jax.nn.silu(g) * u).astype(jnp.bfloat16)
    ys_ref[...] = jnp.dot(a, wd_ref[0], preferred_element_type=jnp.float32)


def _shared_act_body(x_ref, wsg_ref, wsu_ref, a_ref):
    x = x_ref[...]
    g = jnp.dot(x, wsg_ref[...], preferred_element_type=jnp.float32)
    u = jnp.dot(x, wsu_ref[...], preferred_element_type=jnp.float32)
    a_ref[...] = (jax.nn.silu(g) * u).astype(jnp.bfloat16)


def _shared_down_body(a_ref, wsd_ref, gsh_ref, out_ref):
    out_ref[...] = gsh_ref[...] * jnp.dot(a_ref[...], wsd_ref[...],
                                          preferred_element_type=jnp.float32)


def _make_dispatch(T, D, NP, NW, CH):
    mesh = plsc.VectorSubcoreMesh(core_axis_name="c", subcore_axis_name="s")
    tpw = T // NW

    @functools.partial(
        pl.kernel, mesh=mesh,
        out_type=jax.ShapeDtypeStruct((NP, D), jnp.float32),
        scratch_types=[
            pltpu.VMEM((CH, D), jnp.float32),
            pltpu.VMEM((CH,), jnp.int32),
            pltpu.VMEM((CH,), jnp.int32),
            pltpu.SemaphoreType.DMA,
        ],
    )
    def dispatch(x_hbm, p0_hbm, p1_hbm, xs_hbm, rows_v, i0_v, i1_v, sem):
        wid = lax.axis_index("s") * 2 + lax.axis_index("c")
        base = wid * tpw
        for ch in range(0, tpw, CH):
            b = base + ch
            pltpu.sync_copy(x_hbm.at[pl.ds(b, CH)], rows_v)
            pltpu.sync_copy(p0_hbm.at[pl.ds(b, CH)], i0_v)
            pltpu.sync_copy(p1_hbm.at[pl.ds(b, CH)], i1_v)
            c0 = pltpu.async_copy(rows_v, xs_hbm.at[i0_v], sem)
            c1 = pltpu.async_copy(rows_v, xs_hbm.at[i1_v], sem)
            c0.wait()
            c1.wait()

    return dispatch


def _make_combine(T, D, NW, CH):
    mesh = plsc.VectorSubcoreMesh(core_axis_name="c", subcore_axis_name="s")
    tpw = T // NW
    d16 = D // 16

    @functools.partial(
        pl.kernel, mesh=mesh,
        out_type=jax.ShapeDtypeStruct((T, D), jnp.float32),
        scratch_types=[
            pltpu.VMEM((CH, D), jnp.float32),
            pltpu.VMEM((CH, D), jnp.float32),
            pltpu.VMEM((CH, D), jnp.float32),
            pltpu.VMEM((tpw,), jnp.int32),
            pltpu.VMEM((tpw,), jnp.int32),
            pltpu.VMEM((tpw, 16), jnp.float32),
            pltpu.VMEM((tpw, 16), jnp.float32),
            pltpu.SemaphoreType.DMA,
        ],
    )
    def combine(ys_hbm, p0_hbm, p1_hbm, wa_hbm, wb_hbm, sh_hbm, out_hbm,
                y0_v, y1_v, sh_v, p0_v, p1_v, wa_v, wb_v, sem):
        wid = lax.axis_index("s") * 2 + lax.axis_index("c")
        base = wid * tpw
        pltpu.sync_copy(p0_hbm.at[pl.ds(base, tpw)], p0_v)
        pltpu.sync_copy(p1_hbm.at[pl.ds(base, tpw)], p1_v)
        pltpu.sync_copy(wa_hbm.at[pl.ds(base, tpw)], wa_v)
        pltpu.sync_copy(wb_hbm.at[pl.ds(base, tpw)], wb_v)
        for ch in range(0, tpw, CH):
            b = base + ch
            pltpu.async_copy(ys_hbm.at[p0_v.at[pl.ds(ch, CH)]], y0_v, sem).wait()
            pltpu.async_copy(ys_hbm.at[p1_v.at[pl.ds(ch, CH)]], y1_v, sem).wait()
            pltpu.sync_copy(sh_hbm.at[pl.ds(b, CH)], sh_v)

            for r in range(CH):
                was = wa_v[ch + r, :]
                wbs = wb_v[ch + r, :]

                def body(c, _, r=r, was=was, wbs=wbs):
                    o = c * 16
                    v = (was * y0_v[r, pl.ds(o, 16)]
                         + wbs * y1_v[r, pl.ds(o, 16)]
                         + sh_v[r, pl.ds(o, 16)])
                    y0_v[r, pl.ds(o, 16)] = v
                    return 0

                lax.fori_loop(0, d16, body, 0)
            pltpu.sync_copy(y0_v, out_hbm.at[pl.ds(b, CH)])

    return combine


def _pad_ff(w, axis, ffp):
    pad = [(0, 0)] * w.ndim
    pad[axis] = (0, ffp - w.shape[axis])
    return jnp.pad(w, pad)


def kernel(hidden_states, gate_w, w_gate, w_up, w_down, ws_gate, ws_up,
           ws_down, shared_gate_w):
    orig_shape = hidden_states.shape
    D = orig_shape[-1]
    x = hidden_states.reshape(-1, D)
    T = x.shape[0]
    E, _, FF = w_gate.shape
    FFS = ws_gate.shape[1]
    NT = (2 * T) // TILE + E
    NP = NT * TILE

    gwt = jnp.zeros((D, E_PAD), jnp.float32).at[:, :E].set(gate_w.T)
    sgw = shared_gate_w.astype(jnp.float32).reshape(1, D)
    pos0, pos1, wa, wb, te, gsh = pl.pallas_call(
        _router_body,
        out_shape=(
            jax.ShapeDtypeStruct((T, 1), jnp.int32),
            jax.ShapeDtypeStruct((T, 1), jnp.int32),
            jax.ShapeDtypeStruct((T, 16), jnp.float32),
            jax.ShapeDtypeStruct((T, 16), jnp.float32),
            jax.ShapeDtypeStruct((E_PAD, 1), jnp.int32),
            jax.ShapeDtypeStruct((T, 1), jnp.float32),
        ),
    )(x, gwt, sgw)
    pos0 = pos0.reshape(T)
    pos1 = pos1.reshape(T)
    te = te.reshape(E_PAD)

    xb = x.astype(jnp.bfloat16)
    xs = _make_dispatch(T, D, NP, 32, 16)(x, pos0, pos1)

    FFP = ((FF + 511) // 512) * 512
    wg = _pad_ff(w_gate.astype(jnp.bfloat16), 2, FFP)
    wu = _pad_ff(w_up.astype(jnp.bfloat16), 2, FFP)
    wd = _pad_ff(w_down.astype(jnp.bfloat16), 1, FFP)
    ys = pl.pallas_call(
        _grouped_body,
        grid_spec=pltpu.PrefetchScalarGridSpec(
            num_scalar_prefetch=1,
            grid=(NT,),
            in_specs=[
                pl.BlockSpec((TILE, D), lambda i, te_ref: (i, 0)),
                pl.BlockSpec((1, D, FFP), lambda i, te_ref: (te_ref[i], 0, 0)),
                pl.BlockSpec((1, D, FFP), lambda i, te_ref: (te_ref[i], 0, 0)),
                pl.BlockSpec((1, FFP, D), lambda i, te_ref: (te_ref[i], 0, 0)),
            ],
            out_specs=pl.BlockSpec((TILE, D), lambda i, te_ref: (i, 0)),
        ),
        out_shape=jax.ShapeDtypeStruct((NP, D), jnp.float32),
    )(te, xs, wg, wu, wd)

    cs = 512 if FFS % 512 == 0 else FFS
    nj = FFS // cs
    wsg = ws_gate.astype(jnp.bfloat16)
    wsu = ws_up.astype(jnp.bfloat16)
    wsd = ws_down.astype(jnp.bfloat16)
    act = pl.pallas_call(
        _shared_act_body,
        grid=(nj,),
        in_specs=[
            pl.BlockSpec((T, D), lambda j: (0, 0)),
            pl.BlockSpec((D, cs), lambda j: (0, j)),
            pl.BlockSpec((D, cs), lambda j: (0, j)),
        ],
        out_specs=pl.BlockSpec((T, cs), lambda j: (0, j)),
        out_shape=jax.ShapeDtypeStruct((T, FFS), jnp.bfloat16),
    )(xb, wsg, wsu)
    cd = 512 if D % 512 == 0 else D
    nd = D // cd
    sh = pl.pallas_call(
        _shared_down_body,
        grid=(nd,),
        in_specs=[
            pl.BlockSpec((T, FFS), lambda n: (0, 0)),
            pl.BlockSpec((FFS, cd), lambda n: (0, n)),
            pl.BlockSpec((T, 1), lambda n: (0, 0)),
        ],
        out_specs=pl.BlockSpec((T, cd), lambda n: (0, n)),
        out_shape=jax.ShapeDtypeStruct((T, D), jnp.float32),
    )(act, wsd, gsh)

    final = _make_combine(T, D, 32, 8)(ys, pos0, pos1, wa, wb, sh)
    return final.reshape(orig_shape)


# speedup vs baseline: 1.4505x; 1.0056x over previous
import functools

import jax
import jax.numpy as jnp
from jax import lax
from jax.experimental import pallas as pl
from jax.experimental.pallas import tpu as pltpu
from jax.experimental.pallas import tpu_sc as plsc

E_PAD = 128
TILE = 128


def _router_body(x_ref, gwt_ref, sgw_ref, pos0_ref, pos1_ref, wa_ref, wb_ref,
                 te_ref, gsh_ref):
    logits = jnp.dot(x_ref[...], gwt_ref[...], preferred_element_type=jnp.float32)
    T = logits.shape[0]
    gsh_ref[...] = jax.nn.sigmoid(
        jnp.sum(x_ref[...] * sgw_ref[...], axis=1, keepdims=True))
    lane = jax.lax.broadcasted_iota(jnp.int32, (T, E_PAD), 1)
    neg = jnp.float32(-1e30)
    logits = jnp.where(lane < 8, logits, neg)
    m1 = jnp.max(logits, axis=1, keepdims=True)
    i1 = jnp.min(jnp.where(logits == m1, lane, E_PAD), axis=1, keepdims=True)
    masked = jnp.where(lane == i1, neg, logits)
    m2 = jnp.max(masked, axis=1, keepdims=True)
    i2 = jnp.min(jnp.where(masked == m2, lane, E_PAD), axis=1, keepdims=True)
    w1 = 1.0 / (1.0 + jnp.exp(m2 - m1))
    wa_ref[...] = jnp.broadcast_to(w1, wa_ref.shape)
    wb_ref[...] = jnp.broadcast_to(1.0 - w1, wb_ref.shape)

    cnt = (jnp.where(lane == i1, 1.0, 0.0) + jnp.where(lane == i2, 1.0, 0.0))
    ri = jax.lax.broadcasted_iota(jnp.int32, (T, T), 0)
    ci = jax.lax.broadcasted_iota(jnp.int32, (T, T), 1)
    ltri = jnp.where(ci <= ri, 1.0, 0.0).astype(jnp.bfloat16)
    rank_incl = jnp.dot(ltri, cnt.astype(jnp.bfloat16),
                        preferred_element_type=jnp.float32)
    rank0 = rank_incl - cnt
    totals = rank_incl[T - 1:T, :]
    padded = jnp.ceil(totals * (1.0 / TILE)) * TILE
    r128 = jax.lax.broadcasted_iota(jnp.int32, (E_PAD, E_PAD), 0)
    c128 = jax.lax.broadcasted_iota(jnp.int32, (E_PAD, E_PAD), 1)
    upper = jnp.where(r128 < c128, 1.0, 0.0).astype(jnp.bfloat16)
    starts = jnp.dot(padded.astype(jnp.bfloat16), upper,
                     preferred_element_type=jnp.float32)
    ends = starts + padded
    pos0 = jnp.sum(jnp.where(lane == i1, starts + rank0, 0.0), axis=1,
                   keepdims=True)
    pos1 = jnp.sum(jnp.where(lane == i2, starts + rank0, 0.0), axis=1,
                   keepdims=True)
    pos0_ref[...] = pos0.astype(jnp.int32)
    pos1_ref[...] = pos1.astype(jnp.int32)
    rt = jax.lax.broadcasted_iota(jnp.int32, (E_PAD, E_PAD), 0).astype(jnp.float32) * TILE
    elig = (c128 < 8) & (jnp.broadcast_to(ends, (E_PAD, E_PAD)) <= rt)
    te = jnp.sum(jnp.where(elig, 1, 0), axis=1, keepdims=True)
    te_ref[...] = jnp.minimum(te, 7).astype(jnp.int32)


def _grouped_body(te_ref, xs_ref, wg_ref, wu_ref, wd_ref, ys_ref):
    x = xs_ref[...].astype(jnp.bfloat16)
    g = jnp.dot(x, wg_ref[0], preferred_element_type=jnp.float32)
    u = jnp.dot(x, wu_ref[0], preferred_element_type=jnp.float32)
    a = (jax.nn.silu(g) * u).astype(jnp.bfloat16)
    ys_ref[...] = jnp.dot(a, wd_ref[0], preferred_element_type=jnp.float32)


def _shared_act_body(x_ref, wsg_ref, wsu_ref, a_ref):
    x = x_ref[...]
    g = jnp.dot(x, wsg_ref[...], preferred_element_type=jnp.float32)
    u = jnp.dot(x, wsu_ref[...], preferred_element_type=jnp.float32)
    a_ref[...] = (jax.nn.silu(g) * u).astype(jnp.bfloat16)


def _shared_down_body(a_ref, wsd_ref, gsh_ref, out_ref):
    out_ref[...] = gsh_ref[...] * jnp.dot(a_ref[...], wsd_ref[...],
                                          preferred_element_type=jnp.float32)


def _make_dispatch(T, D, NP, NW, CH):
    mesh = plsc.VectorSubcoreMesh(core_axis_name="c", subcore_axis_name="s")
    tpw = T // NW

    @functools.partial(
        pl.kernel, mesh=mesh,
        out_type=jax.ShapeDtypeStruct((NP, D), jnp.float32),
        scratch_types=[
            pltpu.VMEM((CH, D), jnp.float32),
            pltpu.VMEM((CH,), jnp.int32),
            pltpu.VMEM((CH,), jnp.int32),
            pltpu.SemaphoreType.DMA,
        ],
    )
    def dispatch(x_hbm, p0_hbm, p1_hbm, xs_hbm, rows_v, i0_v, i1_v, sem):
        wid = lax.axis_index("s") * 2 + lax.axis_index("c")
        base = wid * tpw
        for ch in range(0, tpw, CH):
            b = base + ch
            pltpu.sync_copy(x_hbm.at[pl.ds(b, CH)], rows_v)
            pltpu.sync_copy(p0_hbm.at[pl.ds(b, CH)], i0_v)
            pltpu.sync_copy(p1_hbm.at[pl.ds(b, CH)], i1_v)
            c0 = pltpu.async_copy(rows_v, xs_hbm.at[i0_v], sem)
            c1 = pltpu.async_copy(rows_v, xs_hbm.at[i1_v], sem)
            c0.wait()
            c1.wait()

    return dispatch


def _make_combine(T, D, NW, CH):
    mesh = plsc.VectorSubcoreMesh(core_axis_name="c", subcore_axis_name="s")
    tpw = T // NW
    d16 = D // 16

    @functools.partial(
        pl.kernel, mesh=mesh,
        out_type=jax.ShapeDtypeStruct((T, D), jnp.float32),
        scratch_types=[
            pltpu.VMEM((CH, D), jnp.float32),
            pltpu.VMEM((CH, D), jnp.float32),
            pltpu.VMEM((CH, D), jnp.float32),
            pltpu.VMEM((tpw,), jnp.int32),
            pltpu.VMEM((tpw,), jnp.int32),
            pltpu.VMEM((tpw, 16), jnp.float32),
            pltpu.VMEM((tpw, 16), jnp.float32),
            pltpu.SemaphoreType.DMA,
        ],
    )
    def combine(ys_hbm, p0_hbm, p1_hbm, wa_hbm, wb_hbm, sh_hbm, out_hbm,
                y0_v, y1_v, sh_v, p0_v, p1_v, wa_v, wb_v, sem):
        wid = lax.axis_index("s") * 2 + lax.axis_index("c")
        base = wid * tpw
        pltpu.sync_copy(p0_hbm.at[pl.ds(base, tpw)], p0_v)
        pltpu.sync_copy(p1_hbm.at[pl.ds(base, tpw)], p1_v)
        pltpu.sync_copy(wa_hbm.at[pl.ds(base, tpw)], wa_v)
        pltpu.sync_copy(wb_hbm.at[pl.ds(base, tpw)], wb_v)
        for ch in range(0, tpw, CH):
            b = base + ch
            pltpu.async_copy(ys_hbm.at[p0_v.at[pl.ds(ch, CH)]], y0_v, sem).wait()
            pltpu.async_copy(ys_hbm.at[p1_v.at[pl.ds(ch, CH)]], y1_v, sem).wait()
            pltpu.sync_copy(sh_hbm.at[pl.ds(b, CH)], sh_v)

            for r in range(CH):
                was = wa_v[ch + r, :]
                wbs = wb_v[ch + r, :]

                def body(c, _, r=r, was=was, wbs=wbs):
                    o = c * 16
                    v = (was * y0_v[r, pl.ds(o, 16)]
                         + wbs * y1_v[r, pl.ds(o, 16)]
                         + sh_v[r, pl.ds(o, 16)])
                    y0_v[r, pl.ds(o, 16)] = v
                    return 0

                lax.fori_loop(0, d16, body, 0)
            pltpu.sync_copy(y0_v, out_hbm.at[pl.ds(b, CH)])

    return combine


def _pad_ff(w, axis, ffp):
    pad = [(0, 0)] * w.ndim
    pad[axis] = (0, ffp - w.shape[axis])
    return jnp.pad(w, pad)


def kernel(hidden_states, gate_w, w_gate, w_up, w_down, ws_gate, ws_up,
           ws_down, shared_gate_w):
    orig_shape = hidden_states.shape
    D = orig_shape[-1]
    x = hidden_states.reshape(-1, D)
    T = x.shape[0]
    E, _, FF = w_gate.shape
    FFS = ws_gate.shape[1]
    NT = (2 * T) // TILE + E
    NP = NT * TILE

    gwt = jnp.zeros((D, E_PAD), jnp.float32).at[:, :E].set(gate_w.T)
    sgw = shared_gate_w.astype(jnp.float32).reshape(1, D)
    pos0, pos1, wa, wb, te, gsh = pl.pallas_call(
        _router_body,
        out_shape=(
            jax.ShapeDtypeStruct((T, 1), jnp.int32),
            jax.ShapeDtypeStruct((T, 1), jnp.int32),
            jax.ShapeDtypeStruct((T, 16), jnp.float32),
            jax.ShapeDtypeStruct((T, 16), jnp.float32),
            jax.ShapeDtypeStruct((E_PAD, 1), jnp.int32),
            jax.ShapeDtypeStruct((T, 1), jnp.float32),
        ),
    )(x, gwt, sgw)
    pos0 = pos0.reshape(T)
    pos1 = pos1.reshape(T)
    te = te.reshape(E_PAD)

    xb = x.astype(jnp.bfloat16)
    xs = _make_dispatch(T, D, NP, 32, 16)(x, pos0, pos1)

    FFP = ((FF + 511) // 512) * 512
    wg = _pad_ff(w_gate.astype(jnp.bfloat16), 2, FFP)
    wu = _pad_ff(w_up.astype(jnp.bfloat16), 2, FFP)
    wd = _pad_ff(w_down.astype(jnp.bfloat16), 1, FFP)
    ys = pl.pallas_call(
        _grouped_body,
        grid_spec=pltpu.PrefetchScalarGridSpec(
            num_scalar_prefetch=1,
            grid=(NT,),
            in_specs=[
                pl.BlockSpec((TILE, D), lambda i, te_ref: (i, 0)),
                pl.BlockSpec((1, D, FFP), lambda i, te_ref: (te_ref[i], 0, 0)),
                pl.BlockSpec((1, D, FFP), lambda i, te_ref: (te_ref[i], 0, 0)),
                pl.BlockSpec((1, FFP, D), lambda i, te_ref: (te_ref[i], 0, 0)),
            ],
            out_specs=pl.BlockSpec((TILE, D), lambda i, te_ref: (i, 0)),
        ),
        out_shape=jax.ShapeDtypeStruct((NP, D), jnp.float32),
    )(te, xs, wg, wu, wd)

    cs = 512 if FFS % 512 == 0 else FFS
    nj = FFS // cs
    wsg = ws_gate.astype(jnp.bfloat16)
    wsu = ws_up.astype(jnp.bfloat16)
    wsd = ws_down.astype(jnp.bfloat16)
    act = pl.pallas_call(
        _shared_act_body,
        grid=(nj,),
        in_specs=[
            pl.BlockSpec((T, D), lambda j: (0, 0)),
            pl.BlockSpec((D, cs), lambda j: (0, j)),
            pl.BlockSpec((D, cs), lambda j: (0, j)),
        ],
        out_specs=pl.BlockSpec((T, cs), lambda j: (0, j)),
        out_shape=jax.ShapeDtypeStruct((T, FFS), jnp.bfloat16),
    )(xb, wsg, wsu)
    cd = 512 if D % 512 == 0 else D
    nd = D // cd
    sh = pl.pallas_call(
        _shared_down_body,
        grid=(nd,),
        in_specs=[
            pl.BlockSpec((T, FFS), lambda n: (0, 0)),
            pl.BlockSpec((FFS, cd), lambda n: (0, n)),
            pl.BlockSpec((T, 1), lambda n: (0, 0)),
        ],
        out_specs=pl.BlockSpec((T, cd), lambda n: (0, n)),
        out_shape=jax.ShapeDtypeStruct((T, D), jnp.float32),
    )(act, wsd, gsh)

    final = _make_combine(T, D, 32, 8)(ys, pos0, pos1, wa, wb, sh)
    return final.reshape(orig_shape)


# speedup vs baseline: 1.5083x; 1.0398x over previous
import functools

import jax
import jax.numpy as jnp
from jax import lax
from jax.experimental import pallas as pl
from jax.experimental.pallas import tpu as pltpu
from jax.experimental.pallas import tpu_sc as plsc

E_PAD = 128
TILE = 128


def _router_body(x_ref, gwt_ref, sgw_ref, pos0_ref, pos1_ref, wa_ref, wb_ref,
                 te_ref, gsh_ref):
    logits = jnp.dot(x_ref[...], gwt_ref[...], preferred_element_type=jnp.float32)
    T = logits.shape[0]
    gsh_ref[...] = jax.nn.sigmoid(
        jnp.sum(x_ref[...] * sgw_ref[...], axis=1, keepdims=True))
    lane = jax.lax.broadcasted_iota(jnp.int32, (T, E_PAD), 1)
    neg = jnp.float32(-1e30)
    logits = jnp.where(lane < 8, logits, neg)
    m1 = jnp.max(logits, axis=1, keepdims=True)
    i1 = jnp.min(jnp.where(logits == m1, lane, E_PAD), axis=1, keepdims=True)
    masked = jnp.where(lane == i1, neg, logits)
    m2 = jnp.max(masked, axis=1, keepdims=True)
    i2 = jnp.min(jnp.where(masked == m2, lane, E_PAD), axis=1, keepdims=True)
    w1 = 1.0 / (1.0 + jnp.exp(m2 - m1))
    wa_ref[...] = jnp.broadcast_to(w1, wa_ref.shape)
    wb_ref[...] = jnp.broadcast_to(1.0 - w1, wb_ref.shape)

    cnt = (jnp.where(lane == i1, 1.0, 0.0) + jnp.where(lane == i2, 1.0, 0.0))
    ri = jax.lax.broadcasted_iota(jnp.int32, (T, T), 0)
    ci = jax.lax.broadcasted_iota(jnp.int32, (T, T), 1)
    ltri = jnp.where(ci <= ri, 1.0, 0.0).astype(jnp.bfloat16)
    rank_incl = jnp.dot(ltri, cnt.astype(jnp.bfloat16),
                        preferred_element_type=jnp.float32)
    rank0 = rank_incl - cnt
    totals = rank_incl[T - 1:T, :]
    padded = jnp.ceil(totals * (1.0 / TILE)) * TILE
    r128 = jax.lax.broadcasted_iota(jnp.int32, (E_PAD, E_PAD), 0)
    c128 = jax.lax.broadcasted_iota(jnp.int32, (E_PAD, E_PAD), 1)
    upper = jnp.where(r128 < c128, 1.0, 0.0).astype(jnp.bfloat16)
    starts = jnp.dot(padded.astype(jnp.bfloat16), upper,
                     preferred_element_type=jnp.float32)
    ends = starts + padded
    pos0 = jnp.sum(jnp.where(lane == i1, starts + rank0, 0.0), axis=1,
                   keepdims=True)
    pos1 = jnp.sum(jnp.where(lane == i2, starts + rank0, 0.0), axis=1,
                   keepdims=True)
    pos0_ref[...] = pos0.astype(jnp.int32)
    pos1_ref[...] = pos1.astype(jnp.int32)
    rt = jax.lax.broadcasted_iota(jnp.int32, (E_PAD, E_PAD), 0).astype(jnp.float32) * TILE
    elig = (c128 < 8) & (jnp.broadcast_to(ends, (E_PAD, E_PAD)) <= rt)
    te = jnp.sum(jnp.where(elig, 1, 0), axis=1, keepdims=True)
    te_ref[...] = jnp.minimum(te, 7).astype(jnp.int32)


def _grouped_body(te_ref, xs_ref, wg_ref, wu_ref, wd_ref, ys_ref):
    x = xs_ref[...].astype(jnp.bfloat16)
    g = jnp.dot(x, wg_ref[0], preferred_element_type=jnp.float32)
    u = jnp.dot(x, wu_ref[0], preferred_element_type=jnp.float32)
    a = (jax.nn.silu(g) * u).astype(jnp.bfloat16)
    ys_ref[...] = jnp.dot(a, wd_ref[0], preferred_element_type=jnp.float32)


def _shared_act_body(x_ref, wsg_ref, wsu_ref, a_ref):
    x = x_ref[...]
    g = jnp.dot(x, wsg_ref[...], preferred_element_type=jnp.float32)
    u = jnp.dot(x, wsu_ref[...], preferred_element_type=jnp.float32)
    a_ref[...] = (jax.nn.silu(g) * u).astype(jnp.bfloat16)


def _shared_down_body(a_ref, wsd_ref, gsh_ref, out_ref):
    out_ref[...] = gsh_ref[...] * jnp.dot(a_ref[...], wsd_ref[...],
                                          preferred_element_type=jnp.float32)


def _make_dispatch(T, D, NP, NW, CH):
    mesh = plsc.VectorSubcoreMesh(core_axis_name="c", subcore_axis_name="s")
    tpw = T // NW

    @functools.partial(
        pl.kernel, mesh=mesh,
        out_type=jax.ShapeDtypeStruct((NP, D), jnp.float32),
        scratch_types=[
            pltpu.VMEM((CH, D), jnp.float32),
            pltpu.VMEM((CH,), jnp.int32),
            pltpu.VMEM((CH,), jnp.int32),
            pltpu.SemaphoreType.DMA,
        ],
    )
    def dispatch(x_hbm, p0_hbm, p1_hbm, xs_hbm, rows_v, i0_v, i1_v, sem):
        wid = lax.axis_index("s") * 2 + lax.axis_index("c")
        base = wid * tpw
        for ch in range(0, tpw, CH):
            b = base + ch
            pltpu.sync_copy(x_hbm.at[pl.ds(b, CH)], rows_v)
            pltpu.sync_copy(p0_hbm.at[pl.ds(b, CH)], i0_v)
            pltpu.sync_copy(p1_hbm.at[pl.ds(b, CH)], i1_v)
            c0 = pltpu.async_copy(rows_v, xs_hbm.at[i0_v], sem)
            c1 = pltpu.async_copy(rows_v, xs_hbm.at[i1_v], sem)
            c0.wait()
            c1.wait()

    return dispatch


def _make_combine(T, D, NW, CH):
    mesh = plsc.VectorSubcoreMesh(core_axis_name="c", subcore_axis_name="s")
    tpw = T // NW
    d16 = D // 16

    @functools.partial(
        pl.kernel, mesh=mesh,
        out_type=jax.ShapeDtypeStruct((T, D), jnp.float32),
        scratch_types=[
            pltpu.VMEM((2, CH, D), jnp.float32),
            pltpu.VMEM((2, CH, D), jnp.float32),
            pltpu.VMEM((2, CH, D), jnp.float32),
            pltpu.VMEM((tpw,), jnp.int32),
            pltpu.VMEM((tpw,), jnp.int32),
            pltpu.VMEM((tpw, 16), jnp.float32),
            pltpu.VMEM((tpw, 16), jnp.float32),
            pltpu.SemaphoreType.DMA,
            pltpu.SemaphoreType.DMA,
        ],
    )
    def combine(ys_hbm, p0_hbm, p1_hbm, wa_hbm, wb_hbm, sh_hbm, out_hbm,
                y0_v, y1_v, sh_v, p0_v, p1_v, wa_v, wb_v, sem0, sem1):
        wid = lax.axis_index("s") * 2 + lax.axis_index("c")
        base = wid * tpw
        pltpu.sync_copy(p0_hbm.at[pl.ds(base, tpw)], p0_v)
        pltpu.sync_copy(p1_hbm.at[pl.ds(base, tpw)], p1_v)
        pltpu.sync_copy(wa_hbm.at[pl.ds(base, tpw)], wa_v)
        pltpu.sync_copy(wb_hbm.at[pl.ds(base, tpw)], wb_v)
        sems = (sem0, sem1)
        nch = tpw // CH

        def fire(k):
            s = k % 2
            return (
                pltpu.async_copy(ys_hbm.at[p0_v.at[pl.ds(k * CH, CH)]],
                                 y0_v.at[s], sems[s]),
                pltpu.async_copy(ys_hbm.at[p1_v.at[pl.ds(k * CH, CH)]],
                                 y1_v.at[s], sems[s]),
                pltpu.async_copy(sh_hbm.at[pl.ds(base + k * CH, CH)],
                                 sh_v.at[s], sems[s]),
            )

        pend = fire(0)
        for k in range(nch):
            s = k % 2
            for c in pend:
                c.wait()
            if k + 1 < nch:
                pend = fire(k + 1)
            for r in range(CH):
                was = wa_v[k * CH + r, :]
                wbs = wb_v[k * CH + r, :]

                def body(c, _, r=r, s=s, was=was, wbs=wbs):
                    o = c * 16
                    v = (was * y0_v[s, r, pl.ds(o, 16)]
                         + wbs * y1_v[s, r, pl.ds(o, 16)]
                         + sh_v[s, r, pl.ds(o, 16)])
                    y0_v[s, r, pl.ds(o, 16)] = v
                    return 0

                lax.fori_loop(0, d16, body, 0)
            pltpu.sync_copy(y0_v.at[s], out_hbm.at[pl.ds(base + k * CH, CH)])

    return combine


def _pad_ff(w, axis, ffp):
    pad = [(0, 0)] * w.ndim
    pad[axis] = (0, ffp - w.shape[axis])
    return jnp.pad(w, pad)


def kernel(hidden_states, gate_w, w_gate, w_up, w_down, ws_gate, ws_up,
           ws_down, shared_gate_w):
    orig_shape = hidden_states.shape
    D = orig_shape[-1]
    x = hidden_states.reshape(-1, D)
    T = x.shape[0]
    E, _, FF = w_gate.shape
    FFS = ws_gate.shape[1]
    NT = (2 * T) // TILE + E
    NP = NT * TILE

    gwt = jnp.zeros((D, E_PAD), jnp.float32).at[:, :E].set(gate_w.T)
    sgw = shared_gate_w.astype(jnp.float32).reshape(1, D)
    pos0, pos1, wa, wb, te, gsh = pl.pallas_call(
        _router_body,
        out_shape=(
            jax.ShapeDtypeStruct((T, 1), jnp.int32),
            jax.ShapeDtypeStruct((T, 1), jnp.int32),
            jax.ShapeDtypeStruct((T, 16), jnp.float32),
            jax.ShapeDtypeStruct((T, 16), jnp.float32),
            jax.ShapeDtypeStruct((E_PAD, 1), jnp.int32),
            jax.ShapeDtypeStruct((T, 1), jnp.float32),
        ),
    )(x, gwt, sgw)
    pos0 = pos0.reshape(T)
    pos1 = pos1.reshape(T)
    te = te.reshape(E_PAD)

    xb = x.astype(jnp.bfloat16)
    xs = _make_dispatch(T, D, NP, 32, 16)(x, pos0, pos1)

    FFP = ((FF + 511) // 512) * 512
    wg = _pad_ff(w_gate.astype(jnp.bfloat16), 2, FFP)
    wu = _pad_ff(w_up.astype(jnp.bfloat16), 2, FFP)
    wd = _pad_ff(w_down.astype(jnp.bfloat16), 1, FFP)
    ys = pl.pallas_call(
        _grouped_body,
        grid_spec=pltpu.PrefetchScalarGridSpec(
            num_scalar_prefetch=1,
            grid=(NT,),
            in_specs=[
                pl.BlockSpec((TILE, D), lambda i, te_ref: (i, 0)),
                pl.BlockSpec((1, D, FFP), lambda i, te_ref: (te_ref[i], 0, 0)),
                pl.BlockSpec((1, D, FFP), lambda i, te_ref: (te_ref[i], 0, 0)),
                pl.BlockSpec((1, FFP, D), lambda i, te_ref: (te_ref[i], 0, 0)),
            ],
            out_specs=pl.BlockSpec((TILE, D), lambda i, te_ref: (i, 0)),
        ),
        out_shape=jax.ShapeDtypeStruct((NP, D), jnp.float32),
    )(te, xs, wg, wu, wd)

    cs = 512 if FFS % 512 == 0 else FFS
    nj = FFS // cs
    wsg = ws_gate.astype(jnp.bfloat16)
    wsu = ws_up.astype(jnp.bfloat16)
    wsd = ws_down.astype(jnp.bfloat16)
    act = pl.pallas_call(
        _shared_act_body,
        grid=(nj,),
        in_specs=[
            pl.BlockSpec((T, D), lambda j: (0, 0)),
            pl.BlockSpec((D, cs), lambda j: (0, j)),
            pl.BlockSpec((D, cs), lambda j: (0, j)),
        ],
        out_specs=pl.BlockSpec((T, cs), lambda j: (0, j)),
        out_shape=jax.ShapeDtypeStruct((T, FFS), jnp.bfloat16),
    )(xb, wsg, wsu)
    cd = 512 if D % 512 == 0 else D
    nd = D // cd
    sh = pl.pallas_call(
        _shared_down_body,
        grid=(nd,),
        in_specs=[
            pl.BlockSpec((T, FFS), lambda n: (0, 0)),
            pl.BlockSpec((FFS, cd), lambda n: (0, n)),
            pl.BlockSpec((T, 1), lambda n: (0, 0)),
        ],
        out_specs=pl.BlockSpec((T, cd), lambda n: (0, n)),
        out_shape=jax.ShapeDtypeStruct((T, D), jnp.float32),
    )(act, wsd, gsh)

    final = _make_combine(T, D, 32, 8)(ys, pos0, pos1, wa, wb, sh)
    return final.reshape(orig_shape)
